# baseline (speedup 1.0000x reference)
import functools

import jax
import jax.numpy as jnp
from jax import lax
from jax.experimental import pallas as pl
from jax.experimental.pallas import tpu as pltpu
from jax.experimental.pallas import tpu_sc as plsc

_ROWS = 8192
_COLS = 768
_TC_ROWS = 6144
_SC_ROWS = _ROWS - _TC_ROWS
_TC_BLOCK_ROWS = 3072

_NUM_WORKERS = 32
_SC_ROWS_PER_WORKER = _SC_ROWS // _NUM_WORKERS

_mesh = plsc.VectorSubcoreMesh(core_axis_name="c", subcore_axis_name="s")


def _tc_body(x_ref, o_ref):
    o_ref[...] = x_ref[...]


@functools.partial(
    pl.kernel,
    mesh=_mesh,
    out_type=jax.ShapeDtypeStruct((_SC_ROWS, _COLS), jnp.float32),
    scratch_types=[pltpu.VMEM((_SC_ROWS_PER_WORKER, _COLS), jnp.float32)],
)
def _sc_copy(x_hbm, o_hbm, buf):
    wid = lax.axis_index("s") * 2 + lax.axis_index("c")
    base = _TC_ROWS + wid * _SC_ROWS_PER_WORKER
    pltpu.sync_copy(x_hbm.at[pl.ds(base, _SC_ROWS_PER_WORKER), :], buf)
    pltpu.sync_copy(buf, o_hbm.at[pl.ds(wid * _SC_ROWS_PER_WORKER, _SC_ROWS_PER_WORKER), :])


def kernel(prototypes):
    top = pl.pallas_call(
        _tc_body,
        out_shape=jax.ShapeDtypeStruct((_TC_ROWS, _COLS), prototypes.dtype),
        grid=(_TC_ROWS // _TC_BLOCK_ROWS,),
        in_specs=[pl.BlockSpec((_TC_BLOCK_ROWS, _COLS), lambda i: (i, 0))],
        out_specs=pl.BlockSpec((_TC_BLOCK_ROWS, _COLS), lambda i: (i, 0)),
        compiler_params=pltpu.CompilerParams(
            dimension_semantics=("parallel",),
        ),
    )(prototypes)
    bottom = _sc_copy(prototypes)
    return jnp.concatenate([top, bottom], axis=0)


# speedup vs baseline: 3.0735x; 3.0735x over previous
import jax
import jax.numpy as jnp
from jax.experimental import pallas as pl
from jax.experimental.pallas import tpu as pltpu

_ROWS = 8192
_COLS = 768
_CHUNKS = 2
_CHUNK_ROWS = _ROWS // _CHUNKS


def _copy_body(x_hbm, o_hbm, buf, in_sems, out_sems):
    def in_copy(c):
        return pltpu.make_async_copy(
            x_hbm.at[pl.ds(c * _CHUNK_ROWS, _CHUNK_ROWS), :],
            buf.at[c],
            in_sems.at[c],
        )

    def out_copy(c):
        return pltpu.make_async_copy(
            buf.at[c],
            o_hbm.at[pl.ds(c * _CHUNK_ROWS, _CHUNK_ROWS), :],
            out_sems.at[c],
        )

    for c in range(_CHUNKS):
        in_copy(c).start()
    for c in range(_CHUNKS):
        in_copy(c).wait()
        out_copy(c).start()
    for c in range(_CHUNKS):
        out_copy(c).wait()


def kernel(prototypes):
    return pl.pallas_call(
        _copy_body,
        out_shape=jax.ShapeDtypeStruct((_ROWS, _COLS), prototypes.dtype),
        in_specs=[pl.BlockSpec(memory_space=pltpu.MemorySpace.HBM)],
        out_specs=pl.BlockSpec(memory_space=pltpu.MemorySpace.HBM),
        scratch_shapes=[
            pltpu.VMEM((_CHUNKS, _CHUNK_ROWS, _COLS), jnp.float32),
            pltpu.SemaphoreType.DMA((_CHUNKS,)),
            pltpu.SemaphoreType.DMA((_CHUNKS,)),
        ],
    )(prototypes)


# speedup vs baseline: 3.1117x; 1.0124x over previous
import jax
import jax.numpy as jnp
from jax.experimental import pallas as pl
from jax.experimental.pallas import tpu as pltpu

_ROWS = 8192
_COLS = 768
_BLOCK_ROWS = 4096


def _copy_body(x_ref, o_ref):
    o_ref[...] = x_ref[...]


def kernel(prototypes):
    return pl.pallas_call(
        _copy_body,
        out_shape=jax.ShapeDtypeStruct((_ROWS, _COLS), prototypes.dtype),
        grid=(_ROWS // _BLOCK_ROWS,),
        in_specs=[pl.BlockSpec((_BLOCK_ROWS, _COLS), lambda i: (i, 0))],
        out_specs=pl.BlockSpec((_BLOCK_ROWS, _COLS), lambda i: (i, 0)),
        compiler_params=pltpu.CompilerParams(
            dimension_semantics=("parallel",),
        ),
    )(prototypes)
